# Initial kernel scaffold; baseline (speedup 1.0000x reference)
import jax
import jax.numpy as jnp
from jax.experimental import pallas as pl


def kernel(x, edge_index, edge_attr, batch, params):
    raise NotImplementedError("write your pallas kernel here")


# speedup vs baseline: 3.1486x; 3.1486x over previous
import functools

import jax
import jax.numpy as jnp
from jax import lax
from jax.experimental import pallas as pl
from jax.experimental.pallas import tpu as pltpu
from jax.experimental.pallas import tpu_sc as plsc

_N = 10000
_E = 320000
_T = 100000
_D = 128
_G = 256

_NC, _NS = 2, 16
_NP = 10240
_RPN = _NP // (_NC * _NS)
_GC = 80

_C = 128
_CHUNKS = _E // _C
_CPS = _CHUNKS // _NC
_KMAX = (_CPS + _NS - 1) // _NS
_ZR = 80
_ZCH = _N // _ZR
_ZK = (_ZCH + _NS - 1) // _NS

_BT = 2048


def _sc_mesh():
    return plsc.VectorSubcoreMesh(
        core_axis_name="c", subcore_axis_name="s",
        num_cores=_NC, num_subcores=_NS)


def _emb_gather(emb, xt):
    @functools.partial(
        pl.kernel,
        out_type=jax.ShapeDtypeStruct((_NP, _D), jnp.float32),
        mesh=_sc_mesh(),
        scratch_types=[
            pltpu.VMEM((_GC,), jnp.int32),
            pltpu.VMEM((_GC, _D), jnp.float32),
            pltpu.SemaphoreType.DMA,
        ],
    )
    def k(emb_hbm, xt_hbm, out_hbm, idx_v, rows_v, sem):
        wid = lax.axis_index("s") * _NC + lax.axis_index("c")

        @pl.loop(0, _RPN // _GC)
        def _(j):
            base = wid * _RPN + j * _GC
            pltpu.sync_copy(xt_hbm.at[pl.ds(base, _GC)], idx_v)
            pltpu.async_copy(emb_hbm.at[idx_v], rows_v, sem).wait()
            pltpu.sync_copy(rows_v, out_hbm.at[pl.ds(base, _GC)])

    return k(emb, xt)


def _edge_aggr(h, src, dst, attr, w, b, zeros):
    @functools.partial(
        pl.kernel,
        out_type=jax.ShapeDtypeStruct((_NC, _N, _D), jnp.float32),
        mesh=_sc_mesh(),
        scratch_types=[
            pltpu.VMEM((_C,), jnp.int32),
            pltpu.VMEM((_C,), jnp.int32),
            pltpu.VMEM((_C,), jnp.float32),
            pltpu.VMEM((_C, _D), jnp.float32),
            pltpu.VMEM((_D,), jnp.float32),
            pltpu.VMEM((_D,), jnp.float32),
            pltpu.VMEM_SHARED((_N, _D), jnp.float32),
            pltpu.SemaphoreType.DMA,
        ],
    )
    def k(h_hbm, src_hbm, dst_hbm, attr_hbm, w_hbm, b_hbm, z_hbm, out_hbm,
          si, di, av, rows, wv, bv, aggr, sem):
        cid = lax.axis_index("c")
        sid = lax.axis_index("s")

        @pl.loop(0, _ZK)
        def _(r):
            m = sid + r * _NS

            @pl.when(m < _ZCH)
            def _():
                pltpu.sync_copy(z_hbm, aggr.at[pl.ds(m * _ZR, _ZR)])

        pltpu.sync_copy(w_hbm, wv)
        pltpu.sync_copy(b_hbm, bv)
        plsc.subcore_barrier()

        wregs = [wv[pl.ds(j * 16, 16)] for j in range(8)]
        bregs = [bv[pl.ds(j * 16, 16)] for j in range(8)]

        @pl.loop(0, _KMAX)
        def _(kk):
            cix = sid + kk * _NS

            @pl.when(cix < _CPS)
            def _():
                off = (cid * _CPS + cix) * _C
                pltpu.sync_copy(src_hbm.at[pl.ds(off, _C)], si)
                pltpu.sync_copy(dst_hbm.at[pl.ds(off, _C)], di)
                pltpu.sync_copy(attr_hbm.at[pl.ds(off, _C)], av)
                pltpu.async_copy(h_hbm.at[si], rows, sem).wait()

                @pl.loop(0, _C // 16)
                def _(g):
                    a16 = av[pl.ds(g * 16, 16)]
                    for i in range(16):
                        a = a16[i]
                        for j in range(8):
                            sl = (g * 16 + i, pl.ds(j * 16, 16))
                            rows[sl] = jnp.maximum(
                                rows[sl] + a * wregs[j] + bregs[j], 0.0)

                pltpu.sync_copy(rows, aggr.at[di], add=True)

        plsc.subcore_barrier()

        @pl.loop(0, _ZK)
        def _(r):
            m = sid + r * _NS

            @pl.when(m < _ZCH)
            def _():
                r0 = m * _ZR
                pltpu.sync_copy(aggr.at[pl.ds(r0, _ZR)],
                                out_hbm.at[cid, pl.ds(r0, _ZR)])

    return k(h, src, dst, attr, w, b, zeros)


def _conv_tc(h, a0, a1, W1, b1, gamma, beta, W2, b2):
    def body(h_ref, a0_ref, a1_ref, w1_ref, b1_ref, g_ref, be_ref,
             w2_ref, b2_ref, out_ref):
        z = h_ref[...] + a0_ref[...] + a1_ref[...]
        z = jnp.dot(z, w1_ref[...], preferred_element_type=jnp.float32)
        z = z + b1_ref[...]
        mu = jnp.mean(z, axis=0, keepdims=True)
        d = z - mu
        var = jnp.mean(d * d, axis=0, keepdims=True)
        z = d * lax.rsqrt(var + 1e-5) * g_ref[...] + be_ref[...]
        z = jnp.maximum(z, 0.0)
        z = jnp.dot(z, w2_ref[...], preferred_element_type=jnp.float32)
        out_ref[...] = jnp.maximum(z + b2_ref[...], 0.0)

    return pl.pallas_call(
        body,
        out_shape=jax.ShapeDtypeStruct((_N, _D), jnp.float32),
    )(h, a0, a1, W1, b1.reshape(1, _D), gamma.reshape(1, _D),
      beta.reshape(1, _D), W2, b2.reshape(1, _D))


def _pool_tc(batch2d, h1, h2):
    def body(b_ref, h1_ref, h2_ref, g1_ref, g2_ref):
        seg = b_ref[...]
        ids = lax.broadcasted_iota(jnp.int32, (_G, _N), 0)
        oh = jnp.where(ids == seg, 1.0, 0.0)
        g1_ref[...] = jnp.dot(oh, h1_ref[...],
                              preferred_element_type=jnp.float32)
        g2_ref[...] = jnp.dot(oh, h2_ref[...],
                              preferred_element_type=jnp.float32)

    return pl.pallas_call(
        body,
        out_shape=[jax.ShapeDtypeStruct((_G, _D), jnp.float32)] * 2,
    )(batch2d, h1, h2)


def _hidden_tc(hg, fW1, fb1, bW1, bb1):
    def body(hg_ref, fw_ref, fb_ref, bw_ref, bb_ref, hf_ref, hb_ref):
        v = hg_ref[...]
        hf_ref[...] = jnp.maximum(
            jnp.dot(v, fw_ref[...], preferred_element_type=jnp.float32)
            + fb_ref[...], 0.0)
        hb_ref[...] = jnp.maximum(
            jnp.dot(v, bw_ref[...], preferred_element_type=jnp.float32)
            + bb_ref[...], 0.0)

    return pl.pallas_call(
        body,
        out_shape=[jax.ShapeDtypeStruct((_G, _D), jnp.float32)] * 2,
    )(hg, fW1, fb1.reshape(1, _D), bW1, bb1.reshape(1, _D))


def _logits_tc(hf, hb, fW2, fb2, bW2, bb2):
    nblk = pl.cdiv(_T, _BT)

    def body(hf_ref, hb_ref, fw_ref, fb_ref, bw_ref, bb_ref, of_ref, ob_ref):
        of_ref[...] = jnp.dot(
            hf_ref[...], fw_ref[...],
            preferred_element_type=jnp.float32) + fb_ref[...]
        ob_ref[...] = jnp.dot(
            hb_ref[...], bw_ref[...],
            preferred_element_type=jnp.float32) + bb_ref[...]

    return pl.pallas_call(
        body,
        grid=(nblk,),
        in_specs=[
            pl.BlockSpec((_G, _D), lambda i: (0, 0)),
            pl.BlockSpec((_G, _D), lambda i: (0, 0)),
            pl.BlockSpec((_D, _BT), lambda i: (0, i)),
            pl.BlockSpec((1, _BT), lambda i: (0, i)),
            pl.BlockSpec((_D, _BT), lambda i: (0, i)),
            pl.BlockSpec((1, _BT), lambda i: (0, i)),
        ],
        out_specs=[
            pl.BlockSpec((_G, _BT), lambda i: (0, i)),
            pl.BlockSpec((_G, _BT), lambda i: (0, i)),
        ],
        out_shape=[jax.ShapeDtypeStruct((_G, _T), jnp.float32)] * 2,
    )(hf, hb, fW2, fb2.reshape(1, _T), bW2, bb2.reshape(1, _T))


def kernel(x, edge_index, edge_attr, batch, params):
    p = params
    src = edge_index[0]
    dst = edge_index[1]
    attr = edge_attr[:, 0]
    xt = jnp.concatenate(
        [x[:, 0], jnp.zeros((_NP - _N,), jnp.int32)])
    zeros = jnp.zeros((_ZR, _D), jnp.float32)

    h0 = _emb_gather(p['emb'], xt)[:_N]
    ag1 = _edge_aggr(h0, src, dst, attr, p['le1_W'][0], p['le1_b'], zeros)
    h1 = _conv_tc(h0, ag1[0], ag1[1], p['c1_W1'], p['c1_b1'],
                  p['c1_gamma'], p['c1_beta'], p['c1_W2'], p['c1_b2'])
    ag2 = _edge_aggr(h1, src, dst, attr, p['le2_W'][0], p['le2_b'], zeros)
    h2 = _conv_tc(h1, ag2[0], ag2[1], p['c2_W1'], p['c2_b1'],
                  p['c2_gamma'], p['c2_beta'], p['c2_W2'], p['c2_b2'])

    g1, g2 = _pool_tc(batch.reshape(1, _N), h1, h2)
    hg = jnp.concatenate([g1, g2], axis=1)
    hf, hb = _hidden_tc(hg, p['f_W1'], p['f_b1'], p['b_W1'], p['b_b1'])
    return tuple(_logits_tc(hf, hb, p['f_W2'], p['f_b2'],
                            p['b_W2'], p['b_b2']))


# speedup vs baseline: 4.8354x; 1.5357x over previous
import dataclasses
import functools

import jax
import jax.numpy as jnp
from jax import lax
from jax.experimental import pallas as pl
from jax.experimental.pallas import tpu as pltpu
from jax.experimental.pallas import tpu_sc as plsc

_N = 10000
_E = 320000
_T = 100000
_D = 128
_G = 256

_NC, _NS = 2, 16
_NP = 10240
_RPN = _NP // (_NC * _NS)
_GC = 80

_C = 128
_CHUNKS = _E // _C
_CPS = _CHUNKS // _NC
_KMAX = (_CPS + _NS - 1) // _NS
_ZR = 80
_ZCH = _N // _ZR
_ZK = (_ZCH + _NS - 1) // _NS

_BT = 2048


def _sc_params():
    cp = pltpu.CompilerParams()
    if "needs_layout_passes" in pltpu.CompilerParams.__dataclass_fields__:
        cp = dataclasses.replace(cp, needs_layout_passes=False)
    return cp


def _sc_mesh():
    return plsc.VectorSubcoreMesh(
        core_axis_name="c", subcore_axis_name="s",
        num_cores=_NC, num_subcores=_NS)


def _emb_gather(emb, xt):
    @functools.partial(
        pl.kernel,
        out_type=jax.ShapeDtypeStruct((_NP, _D), jnp.float32),
        mesh=_sc_mesh(),
        scratch_types=[
            pltpu.VMEM((_GC,), jnp.int32),
            pltpu.VMEM((_GC, _D), jnp.float32),
            pltpu.SemaphoreType.DMA,
        ],
    )
    def k(emb_hbm, xt_hbm, out_hbm, idx_v, rows_v, sem):
        wid = lax.axis_index("s") * _NC + lax.axis_index("c")

        @pl.loop(0, _RPN // _GC)
        def _(j):
            base = wid * _RPN + j * _GC
            pltpu.sync_copy(xt_hbm.at[pl.ds(base, _GC)], idx_v)
            pltpu.async_copy(emb_hbm.at[idx_v], rows_v, sem).wait()
            pltpu.sync_copy(rows_v, out_hbm.at[pl.ds(base, _GC)])

    return k(emb, xt)


def _edge_aggr(h, sda, w, b, zeros):
    @functools.partial(
        pl.kernel,
        out_type=jax.ShapeDtypeStruct((_NC, _N, _D), jnp.float32),
        mesh=_sc_mesh(),
        scratch_types=[
            pltpu.VMEM((3, _C), jnp.int32),
            pltpu.VMEM((3, _C), jnp.int32),
            pltpu.VMEM((3, _C), jnp.int32),
            pltpu.VMEM((_C, _D), jnp.float32),
            pltpu.VMEM((_C, _D), jnp.float32),
            pltpu.VMEM((_C, _D), jnp.float32),
            pltpu.VMEM((_D,), jnp.float32),
            pltpu.VMEM((_D,), jnp.float32),
            pltpu.VMEM_SHARED((_N, _D), jnp.float32),
            pltpu.SemaphoreType.DMA,
            pltpu.SemaphoreType.DMA,
            pltpu.SemaphoreType.DMA,
            pltpu.SemaphoreType.DMA,
            pltpu.SemaphoreType.DMA,
            pltpu.SemaphoreType.DMA,
        ],
        compiler_params=_sc_params(),
    )
    def k(h_hbm, sda_hbm, w_hbm, b_hbm, z_hbm, out_hbm,
          i0, i1, i2, r0, r1, r2, wv, bv, aggr,
          g0, g1, g2, s0, s1, s2):
        ibufs = (i0, i1, i2)
        rows = (r0, r1, r2)
        gsem = (g0, g1, g2)
        ssem = (s0, s1, s2)
        cid = lax.axis_index("c")
        sid = lax.axis_index("s")

        nk = (_CPS - 1 - sid) // _NS + 1

        def chunk_off(k):
            return (cid * _CPS + sid + k * _NS) * _C

        def fetch(k, buf):
            pltpu.sync_copy(sda_hbm.at[:, pl.ds(chunk_off(k), _C)],
                            ibufs[buf])
            pltpu.async_copy(h_hbm.at[ibufs[buf].at[0]], rows[buf],
                             gsem[buf])

        fetch(0, 0)
        pltpu.sync_copy(w_hbm, wv)
        pltpu.sync_copy(b_hbm, bv)

        @pl.loop(0, _ZK)
        def _(r):
            m = sid + r * _NS

            @pl.when(m < _ZCH)
            def _():
                pltpu.sync_copy(z_hbm, aggr.at[pl.ds(m * _ZR, _ZR)])

        plsc.subcore_barrier()

        wregs = [wv[pl.ds(j * 16, 16)] for j in range(8)]
        bregs = [bv[pl.ds(j * 16, 16)] for j in range(8)]

        @pl.loop(0, (_KMAX + 2) // 3)
        def _(q):
            for r in range(3):
                kk = q * 3 + r
                bn = (r + 1) % 3

                @pl.when(kk + 1 < nk)
                def _():
                    @pl.when(kk + 1 >= 3)
                    def _():
                        pltpu.make_async_copy(
                            rows[bn], aggr.at[ibufs[bn].at[1]],
                            ssem[bn]).wait()

                    fetch(kk + 1, bn)

                @pl.when(kk < nk)
                def _():
                    pltpu.make_async_copy(
                        h_hbm.at[ibufs[r].at[0]], rows[r], gsem[r]).wait()

                    @pl.loop(0, _C // 16)
                    def _(g):
                        a16 = plsc.bitcast(
                            ibufs[r][2, pl.ds(g * 16, 16)], jnp.float32)
                        for i in range(16):
                            a = a16[i]
                            for j in range(8):
                                sl = (g * 16 + i, pl.ds(j * 16, 16))
                                rows[r][sl] = jnp.maximum(
                                    rows[r][sl] + a * wregs[j] + bregs[j],
                                    0.0)

                    pltpu.async_copy(rows[r], aggr.at[ibufs[r].at[1]],
                                     ssem[r], add=True)

        for r in range(3):
            @pl.when(nk > r)
            def _():
                pltpu.make_async_copy(
                    rows[r], aggr.at[ibufs[r].at[1]], ssem[r]).wait()

        plsc.subcore_barrier()

        @pl.loop(0, _ZK)
        def _(r):
            m = sid + r * _NS

            @pl.when(m < _ZCH)
            def _():
                r0 = m * _ZR
                pltpu.sync_copy(aggr.at[pl.ds(r0, _ZR)],
                                out_hbm.at[cid, pl.ds(r0, _ZR)])

    return k(h, sda, w, b, zeros)


def _conv_tc(h, a0, a1, W1, b1, gamma, beta, W2, b2):
    def body(h_ref, a0_ref, a1_ref, w1_ref, b1_ref, g_ref, be_ref,
             w2_ref, b2_ref, out_ref):
        z = h_ref[...] + a0_ref[...] + a1_ref[...]
        z = jnp.dot(z, w1_ref[...], preferred_element_type=jnp.float32)
        z = z + b1_ref[...]
        mu = jnp.mean(z, axis=0, keepdims=True)
        d = z - mu
        var = jnp.mean(d * d, axis=0, keepdims=True)
        z = d * lax.rsqrt(var + 1e-5) * g_ref[...] + be_ref[...]
        z = jnp.maximum(z, 0.0)
        z = jnp.dot(z, w2_ref[...], preferred_element_type=jnp.float32)
        out_ref[...] = jnp.maximum(z + b2_ref[...], 0.0)

    return pl.pallas_call(
        body,
        out_shape=jax.ShapeDtypeStruct((_N, _D), jnp.float32),
    )(h, a0, a1, W1, b1.reshape(1, _D), gamma.reshape(1, _D),
      beta.reshape(1, _D), W2, b2.reshape(1, _D))


def _pool_tc(batch2d, h1, h2):
    def body(b_ref, h1_ref, h2_ref, g1_ref, g2_ref):
        seg = b_ref[...]
        ids = lax.broadcasted_iota(jnp.int32, (_G, _N), 0)
        oh = jnp.where(ids == seg, 1.0, 0.0)
        g1_ref[...] = jnp.dot(oh, h1_ref[...],
                              preferred_element_type=jnp.float32)
        g2_ref[...] = jnp.dot(oh, h2_ref[...],
                              preferred_element_type=jnp.float32)

    return pl.pallas_call(
        body,
        out_shape=[jax.ShapeDtypeStruct((_G, _D), jnp.float32)] * 2,
    )(batch2d, h1, h2)


def _hidden_tc(hg, fW1, fb1, bW1, bb1):
    def body(hg_ref, fw_ref, fb_ref, bw_ref, bb_ref, hf_ref, hb_ref):
        v = hg_ref[...]
        hf_ref[...] = jnp.maximum(
            jnp.dot(v, fw_ref[...], preferred_element_type=jnp.float32)
            + fb_ref[...], 0.0)
        hb_ref[...] = jnp.maximum(
            jnp.dot(v, bw_ref[...], preferred_element_type=jnp.float32)
            + bb_ref[...], 0.0)

    return pl.pallas_call(
        body,
        out_shape=[jax.ShapeDtypeStruct((_G, _D), jnp.float32)] * 2,
    )(hg, fW1, fb1.reshape(1, _D), bW1, bb1.reshape(1, _D))


def _logits_tc(hf, hb, fW2, fb2, bW2, bb2):
    nblk = pl.cdiv(_T, _BT)

    def body(hf_ref, hb_ref, fw_ref, fb_ref, bw_ref, bb_ref, of_ref, ob_ref):
        of_ref[...] = jnp.dot(
            hf_ref[...], fw_ref[...],
            preferred_element_type=jnp.float32) + fb_ref[...]
        ob_ref[...] = jnp.dot(
            hb_ref[...], bw_ref[...],
            preferred_element_type=jnp.float32) + bb_ref[...]

    return pl.pallas_call(
        body,
        grid=(nblk,),
        in_specs=[
            pl.BlockSpec((_G, _D), lambda i: (0, 0)),
            pl.BlockSpec((_G, _D), lambda i: (0, 0)),
            pl.BlockSpec((_D, _BT), lambda i: (0, i)),
            pl.BlockSpec((1, _BT), lambda i: (0, i)),
            pl.BlockSpec((_D, _BT), lambda i: (0, i)),
            pl.BlockSpec((1, _BT), lambda i: (0, i)),
        ],
        out_specs=[
            pl.BlockSpec((_G, _BT), lambda i: (0, i)),
            pl.BlockSpec((_G, _BT), lambda i: (0, i)),
        ],
        out_shape=[jax.ShapeDtypeStruct((_G, _T), jnp.float32)] * 2,
    )(hf, hb, fW2, fb2.reshape(1, _T), bW2, bb2.reshape(1, _T))


def kernel(x, edge_index, edge_attr, batch, params):
    p = params
    attr_bits = lax.bitcast_convert_type(edge_attr[:, 0], jnp.int32)
    sda = jnp.concatenate([edge_index, attr_bits[None, :]], axis=0)
    xt = jnp.concatenate(
        [x[:, 0], jnp.zeros((_NP - _N,), jnp.int32)])
    zeros = jnp.zeros((_ZR, _D), jnp.float32)

    h0 = _emb_gather(p['emb'], xt)[:_N]
    ag1 = _edge_aggr(h0, sda, p['le1_W'][0], p['le1_b'], zeros)
    h1 = _conv_tc(h0, ag1[0], ag1[1], p['c1_W1'], p['c1_b1'],
                  p['c1_gamma'], p['c1_beta'], p['c1_W2'], p['c1_b2'])
    ag2 = _edge_aggr(h1, sda, p['le2_W'][0], p['le2_b'], zeros)
    h2 = _conv_tc(h1, ag2[0], ag2[1], p['c2_W1'], p['c2_b1'],
                  p['c2_gamma'], p['c2_beta'], p['c2_W2'], p['c2_b2'])

    g1, g2 = _pool_tc(batch.reshape(1, _N), h1, h2)
    hg = jnp.concatenate([g1, g2], axis=1)
    hf, hb = _hidden_tc(hg, p['f_W1'], p['f_b1'], p['b_W1'], p['b_b1'])
    return tuple(_logits_tc(hf, hb, p['f_W2'], p['f_b2'],
                            p['b_W2'], p['b_b2']))


# speedup vs baseline: 6.2752x; 1.2978x over previous
import dataclasses
import functools

import jax
import jax.numpy as jnp
from jax import lax
from jax.experimental import pallas as pl
from jax.experimental.pallas import tpu as pltpu
from jax.experimental.pallas import tpu_sc as plsc

_N = 10000
_E = 320000
_T = 100000
_D = 128
_G = 256

_NC, _NS = 2, 16
_NP = 10240
_RPN = _NP // (_NC * _NS)
_GC = 80

_C = 128
_CHUNKS = _E // _C
_CPS = _CHUNKS // _NC
_KMAX = (_CPS + _NS - 1) // _NS
_ZR = 80
_ZCH = _N // _ZR
_ZK = (_ZCH + _NS - 1) // _NS

_BT = 2048


def _sc_params():
    cp = pltpu.CompilerParams()
    if "needs_layout_passes" in pltpu.CompilerParams.__dataclass_fields__:
        cp = dataclasses.replace(cp, needs_layout_passes=False)
    return cp


def _sc_mesh():
    return plsc.VectorSubcoreMesh(
        core_axis_name="c", subcore_axis_name="s",
        num_cores=_NC, num_subcores=_NS)


def _emb_gather(emb, xt):
    @functools.partial(
        pl.kernel,
        out_type=jax.ShapeDtypeStruct((_N, _D), jnp.float32),
        mesh=_sc_mesh(),
        scratch_types=[
            pltpu.VMEM((_GC,), jnp.int32),
            pltpu.VMEM((_GC, _D), jnp.float32),
            pltpu.SemaphoreType.DMA,
        ],
    )
    def k(emb_hbm, xt_hbm, out_hbm, idx_v, rows_v, sem):
        wid = lax.axis_index("s") * _NC + lax.axis_index("c")

        @pl.loop(0, _RPN // _GC)
        def _(j):
            base = wid * _RPN + j * _GC

            @pl.when(base < _N)
            def _():
                pltpu.sync_copy(xt_hbm.at[pl.ds(base, _GC)], idx_v)
                pltpu.async_copy(emb_hbm.at[idx_v], rows_v, sem).wait()
                pltpu.sync_copy(rows_v, out_hbm.at[pl.ds(base, _GC)])

    return k(emb, xt)


def _edge_aggr(h, sda, w, b, zeros):
    @functools.partial(
        pl.kernel,
        out_type=jax.ShapeDtypeStruct((_NC, _N, _D), jnp.float32),
        mesh=_sc_mesh(),
        scratch_types=[
            pltpu.VMEM((3, _C), jnp.int32),
            pltpu.VMEM((3, _C), jnp.int32),
            pltpu.VMEM((3, _C), jnp.int32),
            pltpu.VMEM((_C, _D), jnp.float32),
            pltpu.VMEM((_C, _D), jnp.float32),
            pltpu.VMEM((_C, _D), jnp.float32),
            pltpu.VMEM((_D,), jnp.float32),
            pltpu.VMEM((_D,), jnp.float32),
            pltpu.VMEM_SHARED((_N, _D), jnp.float32),
            pltpu.SemaphoreType.DMA,
            pltpu.SemaphoreType.DMA,
            pltpu.SemaphoreType.DMA,
            pltpu.SemaphoreType.DMA,
            pltpu.SemaphoreType.DMA,
            pltpu.SemaphoreType.DMA,
        ],
        compiler_params=_sc_params(),
    )
    def k(h_hbm, sda_hbm, w_hbm, b_hbm, z_hbm, out_hbm,
          i0, i1, i2, r0, r1, r2, wv, bv, aggr,
          g0, g1, g2, s0, s1, s2):
        ibufs = (i0, i1, i2)
        rows = (r0, r1, r2)
        gsem = (g0, g1, g2)
        ssem = (s0, s1, s2)
        cid = lax.axis_index("c")
        sid = lax.axis_index("s")

        nk = (_CPS - 1 - sid) // _NS + 1

        def chunk_off(k):
            return (cid * _CPS + sid + k * _NS) * _C

        def fetch(k, buf):
            pltpu.sync_copy(sda_hbm.at[:, pl.ds(chunk_off(k), _C)],
                            ibufs[buf])
            pltpu.async_copy(h_hbm.at[ibufs[buf].at[0]], rows[buf],
                             gsem[buf])

        fetch(0, 0)
        pltpu.sync_copy(w_hbm, wv)
        pltpu.sync_copy(b_hbm, bv)

        @pl.loop(0, _ZK)
        def _(r):
            m = sid + r * _NS

            @pl.when(m < _ZCH)
            def _():
                pltpu.sync_copy(z_hbm, aggr.at[pl.ds(m * _ZR, _ZR)])

        plsc.subcore_barrier()

        wregs = [wv[pl.ds(j * 16, 16)] for j in range(8)]
        bregs = [bv[pl.ds(j * 16, 16)] for j in range(8)]

        @pl.loop(0, (_KMAX + 2) // 3)
        def _(q):
            for r in range(3):
                kk = q * 3 + r
                bn = (r + 1) % 3

                @pl.when(kk + 1 < nk)
                def _():
                    @pl.when(kk + 1 >= 3)
                    def _():
                        pltpu.make_async_copy(
                            rows[bn], aggr.at[ibufs[bn].at[1]],
                            ssem[bn]).wait()

                    fetch(kk + 1, bn)

                @pl.when(kk < nk)
                def _():
                    pltpu.make_async_copy(
                        h_hbm.at[ibufs[r].at[0]], rows[r], gsem[r]).wait()

                    @pl.loop(0, _C // 16)
                    def _(g):
                        a16 = plsc.bitcast(
                            ibufs[r][2, pl.ds(g * 16, 16)], jnp.float32)
                        for i in range(16):
                            a = a16[i]
                            for j in range(8):
                                sl = (g * 16 + i, pl.ds(j * 16, 16))
                                rows[r][sl] = jnp.maximum(
                                    rows[r][sl] + a * wregs[j] + bregs[j],
                                    0.0)

                    pltpu.async_copy(rows[r], aggr.at[ibufs[r].at[1]],
                                     ssem[r], add=True)

        for r in range(3):
            @pl.when(nk > r)
            def _():
                pltpu.make_async_copy(
                    rows[r], aggr.at[ibufs[r].at[1]], ssem[r]).wait()

        plsc.subcore_barrier()

        @pl.loop(0, _ZK)
        def _(r):
            m = sid + r * _NS

            @pl.when(m < _ZCH)
            def _():
                r0 = m * _ZR
                pltpu.sync_copy(aggr.at[pl.ds(r0, _ZR)],
                                out_hbm.at[cid, pl.ds(r0, _ZR)])

    return k(h, sda, w, b, zeros)


def _conv_tc(h, a0, a1, W1, b1, gamma, beta, W2, b2):
    def body(h_ref, a0_ref, a1_ref, w1_ref, b1_ref, g_ref, be_ref,
             w2_ref, b2_ref, out_ref):
        z = h_ref[...] + a0_ref[...] + a1_ref[...]
        z = jnp.dot(z, w1_ref[...], preferred_element_type=jnp.float32)
        z = z + b1_ref[...]
        mu = jnp.mean(z, axis=0, keepdims=True)
        d = z - mu
        var = jnp.mean(d * d, axis=0, keepdims=True)
        z = d * lax.rsqrt(var + 1e-5) * g_ref[...] + be_ref[...]
        z = jnp.maximum(z, 0.0)
        z = jnp.dot(z, w2_ref[...], preferred_element_type=jnp.float32)
        out_ref[...] = jnp.maximum(z + b2_ref[...], 0.0)

    return pl.pallas_call(
        body,
        out_shape=jax.ShapeDtypeStruct((_N, _D), jnp.float32),
    )(h, a0, a1, W1, b1.reshape(1, _D), gamma.reshape(1, _D),
      beta.reshape(1, _D), W2, b2.reshape(1, _D))


def _pool_tc(batch2d, h1, h2):
    def body(b_ref, h1_ref, h2_ref, g1_ref, g2_ref):
        seg = b_ref[...]
        ids = lax.broadcasted_iota(jnp.int32, (_G, _N), 0)
        oh = jnp.where(ids == seg, 1.0, 0.0)
        g1_ref[...] = jnp.dot(oh, h1_ref[...],
                              preferred_element_type=jnp.float32)
        g2_ref[...] = jnp.dot(oh, h2_ref[...],
                              preferred_element_type=jnp.float32)

    return pl.pallas_call(
        body,
        out_shape=[jax.ShapeDtypeStruct((_G, _D), jnp.float32)] * 2,
    )(batch2d, h1, h2)


def _hidden_tc(hg, fW1, fb1, bW1, bb1):
    def body(hg_ref, fw_ref, fb_ref, bw_ref, bb_ref, hf_ref, hb_ref):
        v = hg_ref[...]
        hf_ref[...] = jnp.maximum(
            jnp.dot(v, fw_ref[...], preferred_element_type=jnp.float32)
            + fb_ref[...], 0.0)
        hb_ref[...] = jnp.maximum(
            jnp.dot(v, bw_ref[...], preferred_element_type=jnp.float32)
            + bb_ref[...], 0.0)

    return pl.pallas_call(
        body,
        out_shape=[jax.ShapeDtypeStruct((_G, _D), jnp.float32)] * 2,
    )(hg, fW1, fb1.reshape(1, _D), bW1, bb1.reshape(1, _D))


def _logits_tc(hfT, hbT, fW2T, fb2, bW2T, bb2):
    nblk = pl.cdiv(_T, _BT)

    def body(hf_ref, hb_ref, fw_ref, fb_ref, bw_ref, bb_ref, of_ref, ob_ref):
        of_ref[...] = jnp.dot(
            fw_ref[...], hf_ref[...],
            preferred_element_type=jnp.float32) + fb_ref[...]
        ob_ref[...] = jnp.dot(
            bw_ref[...], hb_ref[...],
            preferred_element_type=jnp.float32) + bb_ref[...]

    return pl.pallas_call(
        body,
        grid=(nblk,),
        in_specs=[
            pl.BlockSpec((_D, _G), lambda i: (0, 0)),
            pl.BlockSpec((_D, _G), lambda i: (0, 0)),
            pl.BlockSpec((_BT, _D), lambda i: (i, 0)),
            pl.BlockSpec((_BT, 1), lambda i: (i, 0)),
            pl.BlockSpec((_BT, _D), lambda i: (i, 0)),
            pl.BlockSpec((_BT, 1), lambda i: (i, 0)),
        ],
        out_specs=[
            pl.BlockSpec((_BT, _G), lambda i: (i, 0)),
            pl.BlockSpec((_BT, _G), lambda i: (i, 0)),
        ],
        out_shape=[jax.ShapeDtypeStruct((_T, _G), jnp.float32)] * 2,
    )(hfT, hbT, fW2T, fb2.reshape(_T, 1), bW2T, bb2.reshape(_T, 1))


def kernel(x, edge_index, edge_attr, batch, params):
    p = params
    attr_bits = lax.bitcast_convert_type(edge_attr[:, 0], jnp.int32)
    sda = jnp.concatenate([edge_index, attr_bits[None, :]], axis=0)
    xt = jnp.concatenate(
        [x[:, 0], jnp.zeros((_NP - _N,), jnp.int32)])
    zeros = jnp.zeros((_ZR, _D), jnp.float32)

    h0 = _emb_gather(p['emb'], xt)
    ag1 = _edge_aggr(h0, sda, p['le1_W'][0], p['le1_b'], zeros)
    h1 = _conv_tc(h0, ag1[0], ag1[1], p['c1_W1'], p['c1_b1'],
                  p['c1_gamma'], p['c1_beta'], p['c1_W2'], p['c1_b2'])
    ag2 = _edge_aggr(h1, sda, p['le2_W'][0], p['le2_b'], zeros)
    h2 = _conv_tc(h1, ag2[0], ag2[1], p['c2_W1'], p['c2_b1'],
                  p['c2_gamma'], p['c2_beta'], p['c2_W2'], p['c2_b2'])

    g1, g2 = _pool_tc(batch.reshape(1, _N), h1, h2)
    hg = jnp.concatenate([g1, g2], axis=1)
    hf, hb = _hidden_tc(hg, p['f_W1'], p['f_b1'], p['b_W1'], p['b_b1'])
    lfT, lbT = _logits_tc(hf.T, hb.T, p['f_W2'].T, p['f_b2'],
                          p['b_W2'].T, p['b_b2'])
    return (lfT.T, lbT.T)
